# baseline (speedup 1.0000x reference)
import jax
import jax.numpy as jnp
from jax.experimental import pallas as pl
from jax.experimental.pallas import tpu as pltpu

N_EXPERTS = 16
TOP_K = 2
D_MODEL = 2048
N_TOKENS = 16384

BLK = 1024
HALF = N_TOKENS // 2
N_STEPS = HALF // BLK


def _route_block(logits):
    e_iota = jax.lax.broadcasted_iota(jnp.int32, logits.shape, 1)
    big = jnp.int32(N_EXPERTS)
    m0 = jnp.max(logits, axis=1, keepdims=True)
    i0 = jnp.min(jnp.where(logits == m0, e_iota, big), axis=1, keepdims=True)
    masked = jnp.where(e_iota == i0, -jnp.inf, logits)
    m1 = jnp.max(masked, axis=1, keepdims=True)
    i1 = jnp.min(jnp.where(masked == m1, e_iota, big), axis=1, keepdims=True)
    e = jnp.exp(m1 - m0)
    w0 = 1.0 / (1.0 + e)
    w1 = e * w0
    rm = jnp.where(e_iota == i0, w0,
                   jnp.where(e_iota == i1, w1, jnp.float32(0.0)))
    return rm, jnp.concatenate([i0, i1], axis=1)


def _router_body(t_ref, xa_ref, xb_ref, w_ref, rm_ref, idx_ref):
    inv_t = 1.0 / t_ref[0]
    dn = (((1,), (1,)), ((), ()))
    w = w_ref[...]
    logits_a = jax.lax.dot_general(
        xa_ref[...], w, dimension_numbers=dn,
        preferred_element_type=jnp.float32) * inv_t
    logits_b = jax.lax.dot_general(
        xb_ref[...], w, dimension_numbers=dn,
        preferred_element_type=jnp.float32) * inv_t
    rm_a, idx_a = _route_block(logits_a)
    rm_b, idx_b = _route_block(logits_b)
    rm_ref[0] = rm_a
    rm_ref[1] = rm_b
    idx_ref[0] = idx_a
    idx_ref[1] = idx_b


def kernel(x, W, temperature):
    t = jnp.asarray(temperature, jnp.float32).reshape(1)
    rm, idx = pl.pallas_call(
        _router_body,
        grid=(N_STEPS,),
        in_specs=[
            pl.BlockSpec(memory_space=pltpu.SMEM),
            pl.BlockSpec((BLK, D_MODEL), lambda i: (i, 0)),
            pl.BlockSpec((BLK, D_MODEL), lambda i: (i + N_STEPS, 0)),
            pl.BlockSpec((N_EXPERTS, D_MODEL), lambda i: (0, 0)),
        ],
        out_specs=[
            pl.BlockSpec((2, BLK, N_EXPERTS), lambda i: (0, i, 0)),
            pl.BlockSpec((2, BLK, TOP_K), lambda i: (0, i, 0)),
        ],
        out_shape=[
            jax.ShapeDtypeStruct((2, HALF, N_EXPERTS), jnp.float32),
            jax.ShapeDtypeStruct((2, HALF, TOP_K), jnp.int32),
        ],
        compiler_params=pltpu.CompilerParams(
            dimension_semantics=("arbitrary",),
        ),
    )(t, x, x, W)
    return (rm.reshape(N_TOKENS, N_EXPERTS), idx.reshape(N_TOKENS, TOP_K))
